# baseline (speedup 1.0000x reference)
import functools

import jax
import jax.numpy as jnp
from jax import lax
from jax.experimental import pallas as pl
from jax.experimental.pallas import tpu as pltpu
from jax.experimental.pallas import tpu_sc as plsc

SC_CORES = 2
SC_SUBCORES = 16
NW = SC_CORES * SC_SUBCORES
LANE = 16


def _lane_bcast(v, h):
    idx = jnp.full((LANE, 1), h, jnp.int32)
    dn = lax.GatherDimensionNumbers(
        offset_dims=(), collapsed_slice_dims=(0,), start_index_map=(0,))
    return lax.gather(v, idx, dn, (1,),
                      mode=lax.GatherScatterMode.PROMISE_IN_BOUNDS)


def _make_edge_pass(n, ep, fr, aw, heads, ch, blk, sub, mode):
    nv_acc = aw // LANE
    mesh = plsc.VectorSubcoreMesh(
        core_axis_name="c", subcore_axis_name="s",
        num_cores=SC_CORES, num_subcores=SC_SUBCORES)

    ep_max = ep - blk

    def body(feat, ssrc, sdst, srcp, dstp, wb, out,
             wbv, sidx0, didx0, dpad0, rows0, srows0, drows0,
             wbuf, acc, gsem0):
        sidx = (sidx0,)
        didx = (didx0,)
        dpad = (dpad0,)
        rows = (rows0,)
        srows = (srows0,)
        drows = (drows0,)
        gsem = (gsem0,)
        cid = lax.axis_index("c")
        sid = lax.axis_index("s")
        wid = sid * SC_CORES + cid
        pltpu.sync_copy(wb.at[wid], wbv)
        wbvec = wbv[...]
        a0 = wbvec[0]
        e_lo = wbvec[1]
        e_hi = wbvec[2]
        d0 = wbvec[3]
        zv = jnp.zeros((LANE,), jnp.float32)
        for v in range(nv_acc):
            acc[pl.ds(LANE * v, LANE)] = zv
        dpad[0][pl.ds(blk, LANE)] = jnp.zeros((LANE,), jnp.int32)

        def e_of(c):
            return pl.multiple_of(jnp.minimum(a0 + c * blk, ep_max), 8)

        def fetch_idx(b, c):
            e0 = e_of(c)
            pltpu.sync_copy(srcp.at[pl.ds(e0, blk)], sidx[b])
            pltpu.sync_copy(dstp.at[pl.ds(e0, blk)], didx[b])
            pltpu.sync_copy(dstp.at[pl.ds(e0, blk)],
                            dpad[b].at[pl.ds(0, blk)])

        def issue(b):
            pltpu.async_copy(feat.at[sidx[b]], rows[b], gsem[b])
            pltpu.async_copy(ssrc.at[sidx[b]], srows[b], gsem[b])
            pltpu.async_copy(sdst.at[didx[b]], drows[b], gsem[b])

        def drain(b):
            pltpu.make_async_copy(feat.at[sidx[b]], rows[b], gsem[b]).wait()
            pltpu.make_async_copy(ssrc.at[sidx[b]], srows[b],
                                  gsem[b]).wait()
            pltpu.make_async_copy(sdst.at[didx[b]], drows[b],
                                  gsem[b]).wait()

        def compute(b, c, cur_d):
            e0r = a0 + c * blk

            def wstep(k, carry):
                e = (srows[b][k, pl.ds(0, LANE)]
                     + drows[b][k, pl.ds(0, LANE)])
                e = jnp.where(e >= 0.0, e, 0.2 * e)
                w = jnp.exp(e)
                ge = e0r + k
                ok = jnp.logical_and(ge >= e_lo, ge < e_hi)
                wbuf[k] = jnp.where(ok, w, 0.0)
                return carry

            lax.fori_loop(0, blk, wstep, 0)

            def estep(s, cur_d):
                k0 = s * sub
                for kk in range(sub):
                    k = k0 + kk
                    dk = dpad[b][pl.ds(k, LANE)][0]
                    ge = e0r + k
                    ok = jnp.logical_and(ge >= e_lo, ge < e_hi)
                    dk = jnp.where(ok, dk, cur_d)

                    @pl.when(dk != cur_d)
                    def _(cur_d=cur_d):
                        pltpu.sync_copy(acc, out.at[cur_d])
                        for v in range(nv_acc):
                            acc[pl.ds(LANE * v, LANE)] = zv

                    wv = wbuf[k]
                    if mode == "outer":
                        xr = [rows[b][k, pl.ds(LANE * j, LANE)]
                              for j in range(fr // LANE)]
                        for h in range(heads):
                            bh = _lane_bcast(wv, h)
                            for j in range(fr // LANE):
                                plsc.addupdate(
                                    acc.at[pl.ds(h * fr + LANE * j, LANE)],
                                    bh * xr[j])
                    elif mode == "flat":
                        for h in range(heads):
                            bh = _lane_bcast(wv, h)
                            for j in range(ch // LANE):
                                o = h * ch + LANE * j
                                plsc.addupdate(
                                    acc.at[pl.ds(o, LANE)],
                                    bh * rows[b][k, pl.ds(o, LANE)])
                    else:
                        plsc.addupdate(acc.at[pl.ds(0, LANE)],
                                       wv * rows[b][k, pl.ds(0, LANE)])
                    plsc.addupdate(acc.at[pl.ds(aw - LANE, LANE)], wv)
                    cur_d = dk
                return cur_d

            return lax.fori_loop(0, blk // sub, estep, cur_d)

        fetch_idx(0, 0)
        nch = (e_hi - a0 + (blk - 1)) // blk

        def chunkbody(c, cur_d):
            issue(0)
            drain(0)
            cur_d = compute(0, c, cur_d)
            fetch_idx(0, c + 1)
            return cur_d

        cur_d = lax.fori_loop(0, nch, chunkbody, d0)
        pltpu.sync_copy(acc, out.at[cur_d])

    return pl.kernel(
        body,
        out_type=jax.ShapeDtypeStruct((n, aw), jnp.float32),
        mesh=mesh,
        scratch_types=[
            pltpu.VMEM((LANE,), jnp.int32),
            pltpu.VMEM((blk,), jnp.int32),
            pltpu.VMEM((blk,), jnp.int32),
            pltpu.VMEM((blk + LANE,), jnp.int32),
            pltpu.VMEM((blk, fr), jnp.float32),
            pltpu.VMEM((blk, 128), jnp.float32),
            pltpu.VMEM((blk, 128), jnp.float32),
            pltpu.VMEM((blk, LANE), jnp.float32),
            pltpu.VMEM((aw,), jnp.float32),
            pltpu.SemaphoreType.DMA,
        ],
    )


def _prep_edges(edge_index, n, blk_max):
    e = edge_index.shape[1]
    loop = jnp.arange(n, dtype=jnp.int32)
    src = jnp.concatenate([edge_index[0].astype(jnp.int32), loop])
    dst = jnp.concatenate([edge_index[1].astype(jnp.int32), loop])
    order = jnp.argsort(dst)
    src_s = jnp.take(src, order)
    dst_s = jnp.take(dst, order)
    et = e + n
    ep = ((et + blk_max + 7) // 8) * 8
    pad = ep - et
    src_p = jnp.concatenate([src_s, jnp.zeros((pad,), jnp.int32)])
    dst_p = jnp.concatenate([dst_s, jnp.zeros((pad,), jnp.int32)])
    off = jnp.searchsorted(dst_s, jnp.arange(n + 1)).astype(jnp.int32)
    rpw = -(-n // NW)
    wstart = jnp.minimum(jnp.arange(NW, dtype=jnp.int32) * rpw, n)
    wend = jnp.minimum(wstart + rpw, n)
    e_lo = jnp.take(off, wstart)
    e_hi = jnp.take(off, wend)
    a0 = (e_lo // 8) * 8
    wb = jnp.zeros((NW, LANE), jnp.int32)
    wb = (wb.at[:, 0].set(a0).at[:, 1].set(e_lo)
            .at[:, 2].set(e_hi).at[:, 3].set(wstart))
    return src_p, dst_p, wb


def _wcat(w, a_s, a_d):
    din = w.shape[0]
    h, c = a_s.shape
    wr = w.reshape(din, h, c)
    ws = jnp.einsum("dhc,hc->dh", wr, a_s)
    wd = jnp.einsum("dhc,hc->dh", wr, a_d)
    if h == 1:
        return jnp.concatenate(
            [jnp.tile(ws, (1, LANE)), jnp.tile(wd, (1, LANE))], axis=1)
    z = jnp.zeros((din, LANE - h), jnp.float32)
    return jnp.concatenate([ws, z, wd, z], axis=1)


def _tc_scores(x, wcat):
    n, d = x.shape
    br = 512
    grid = pl.cdiv(n, br)

    def body(x_ref, w_ref, s_ref, d_ref):
        s = jnp.dot(x_ref[...], w_ref[...],
                    preferred_element_type=jnp.float32)
        z = jnp.zeros((s.shape[0], 128 - LANE), jnp.float32)
        s_ref[...] = jnp.concatenate([s[:, :LANE], z], axis=1)
        d_ref[...] = jnp.concatenate([s[:, LANE:2 * LANE], z], axis=1)

    return pl.pallas_call(
        body,
        grid=(grid,),
        in_specs=[pl.BlockSpec((br, d), lambda i: (i, 0)),
                  pl.BlockSpec((d, 2 * LANE), lambda i: (0, 0))],
        out_specs=[pl.BlockSpec((br, 128), lambda i: (i, 0)),
                   pl.BlockSpec((br, 128), lambda i: (i, 0))],
        out_shape=[jax.ShapeDtypeStruct((n, 128), jnp.float32)] * 2,
    )(x, wcat)


def _tc_layer1(nd1, w1, b1, g1, be1, w2, wcat2, heads, d_in, c1):
    n = nd1.shape[0]
    f = heads * c1
    hd = heads * d_in
    br = 256
    grid = pl.cdiv(n, br)

    def body(num_ref, den_ref, w1_ref, b1_ref, g1_ref, be1_ref,
             w2_ref, wc_ref, hh_ref, s_ref, d_ref):
        den = den_ref[...][:, :LANE]
        num = num_ref[...]
        parts = []
        for h in range(heads):
            ah = num[:, h * d_in:(h + 1) * d_in] / den[:, h:h + 1]
            parts.append(jnp.dot(ah, w1_ref[:, h * c1:(h + 1) * c1],
                                 preferred_element_type=jnp.float32))
        out1 = jnp.concatenate(parts, axis=1) + b1_ref[...]
        act = jnp.maximum(g1_ref[...] * out1 + be1_ref[...], 0.0)
        hh_ref[...] = jnp.dot(act, w2_ref[...],
                              preferred_element_type=jnp.float32)
        s = jnp.dot(act, wc_ref[...], preferred_element_type=jnp.float32)
        z = jnp.zeros((s.shape[0], 128 - LANE), jnp.float32)
        s_ref[...] = jnp.concatenate([s[:, :LANE], z], axis=1)
        d_ref[...] = jnp.concatenate([s[:, LANE:2 * LANE], z], axis=1)

    return pl.pallas_call(
        body,
        grid=(grid,),
        in_specs=[
            pl.BlockSpec((br, hd), lambda i: (i, 0)),
            pl.BlockSpec((br, 128), lambda i: (i, hd // 128)),
            pl.BlockSpec((d_in, f), lambda i: (0, 0)),
            pl.BlockSpec((1, f), lambda i: (0, 0)),
            pl.BlockSpec((1, f), lambda i: (0, 0)),
            pl.BlockSpec((1, f), lambda i: (0, 0)),
            pl.BlockSpec((f, f), lambda i: (0, 0)),
            pl.BlockSpec((f, 2 * LANE), lambda i: (0, 0)),
        ],
        out_specs=[pl.BlockSpec((br, f), lambda i: (i, 0)),
                   pl.BlockSpec((br, 128), lambda i: (i, 0)),
                   pl.BlockSpec((br, 128), lambda i: (i, 0))],
        out_shape=[jax.ShapeDtypeStruct((n, f), jnp.float32),
                   jax.ShapeDtypeStruct((n, 128), jnp.float32),
                   jax.ShapeDtypeStruct((n, 128), jnp.float32)],
    )(nd1, nd1, w1, b1.reshape(1, f), g1.reshape(1, f),
      be1.reshape(1, f), w2, wcat2)


def _tc_layer2(nd2, b2, g2, be2, w3p, wcat3, heads, c1):
    n = nd2.shape[0]
    f = heads * c1
    br = 256
    grid = pl.cdiv(n, br)

    def body(num_ref, den_ref, b2_ref, g2_ref, be2_ref,
             w3_ref, wc_ref, hh_ref, s_ref, d_ref):
        den = den_ref[...][:, :LANE]
        num = num_ref[...]
        parts = []
        for h in range(heads):
            sl = slice(h * c1, (h + 1) * c1)
            oh = num[:, sl] / den[:, h:h + 1] + b2_ref[:, sl]
            parts.append(jnp.maximum(g2_ref[:, sl] * oh + be2_ref[:, sl],
                                     0.0))
        act = jnp.concatenate(parts, axis=1)
        hh_ref[...] = jnp.dot(act, w3_ref[...],
                              preferred_element_type=jnp.float32)
        s = jnp.dot(act, wc_ref[...], preferred_element_type=jnp.float32)
        z = jnp.zeros((s.shape[0], 128 - LANE), jnp.float32)
        s_ref[...] = jnp.concatenate([s[:, :LANE], z], axis=1)
        d_ref[...] = jnp.concatenate([s[:, LANE:2 * LANE], z], axis=1)

    return pl.pallas_call(
        body,
        grid=(grid,),
        in_specs=[
            pl.BlockSpec((br, f), lambda i: (i, 0)),
            pl.BlockSpec((br, 128), lambda i: (i, f // 128)),
            pl.BlockSpec((1, f), lambda i: (0, 0)),
            pl.BlockSpec((1, f), lambda i: (0, 0)),
            pl.BlockSpec((1, f), lambda i: (0, 0)),
            pl.BlockSpec((f, 128), lambda i: (0, 0)),
            pl.BlockSpec((f, 2 * LANE), lambda i: (0, 0)),
        ],
        out_specs=[pl.BlockSpec((br, 128), lambda i: (i, 0)),
                   pl.BlockSpec((br, 128), lambda i: (i, 0)),
                   pl.BlockSpec((br, 128), lambda i: (i, 0))],
        out_shape=[jax.ShapeDtypeStruct((n, 128), jnp.float32),
                   jax.ShapeDtypeStruct((n, 128), jnp.float32),
                   jax.ShapeDtypeStruct((n, 128), jnp.float32)],
    )(nd2, nd2, b2.reshape(1, f), g2.reshape(1, f), be2.reshape(1, f),
      w3p, wcat3)


def _tc_logsoftmax(nd3, b3p, ncls):
    n = nd3.shape[0]
    br = 512
    grid = pl.cdiv(n, br)

    def body(nd_ref, b3_ref, out_ref):
        nd = nd_ref[...]
        h = nd[:, :LANE] / nd[:, LANE:2 * LANE] + b3_ref[...]
        col = lax.broadcasted_iota(jnp.int32, h.shape, 1)
        valid = col < ncls
        hm = jnp.where(valid, h, jnp.float32(-1e30))
        m = jnp.max(hm, axis=1, keepdims=True)
        ex = jnp.where(valid, jnp.exp(h - m), 0.0)
        s = jnp.sum(ex, axis=1, keepdims=True)
        out_ref[...] = h - m - jnp.log(s)

    return pl.pallas_call(
        body,
        grid=(grid,),
        in_specs=[pl.BlockSpec((br, 2 * LANE), lambda i: (i, 0)),
                  pl.BlockSpec((1, LANE), lambda i: (0, 0))],
        out_specs=pl.BlockSpec((br, LANE), lambda i: (i, 0)),
        out_shape=jax.ShapeDtypeStruct((n, LANE), jnp.float32),
    )(nd3, b3p)


@jax.jit
def kernel(x, edge_index, W1, a1s, a1d, b1, g1, be1,
           W2, a2s, a2d, b2, g2, be2, W3, a3s, a3d, b3):
    n, d = x.shape
    heads, c1 = a1s.shape
    f = heads * c1
    ncls = W3.shape[1]

    blk1, blk2, blk3 = 128, 48, 128
    src_p, dst_p, wb = _prep_edges(edge_index, n, max(blk1, blk2, blk3))
    ep = src_p.shape[0]

    s1, d1 = _tc_scores(x, _wcat(W1, a1s, a1d))
    aw1 = heads * d + LANE
    nd1 = _make_edge_pass(n, ep, d, aw1, heads, d, blk1, 16, "outer")(
        x, s1, d1, src_p, dst_p, wb)

    hh2, s2, d2 = _tc_layer1(nd1, W1, b1, g1, be1, W2,
                             _wcat(W2, a2s, a2d), heads, d, c1)

    aw2 = f + LANE
    nd2 = _make_edge_pass(n, ep, f, aw2, heads, c1, blk2, 8, "flat")(
        hh2, s2, d2, src_p, dst_p, wb)

    w3p = jnp.concatenate(
        [W3, jnp.zeros((f, 128 - ncls), jnp.float32)], axis=1)
    hh3, s3, d3 = _tc_layer2(nd2, b2, g2, be2, w3p,
                             _wcat(W3, a3s, a3d), heads, c1)

    nd3 = _make_edge_pass(n, ep, 128, 2 * LANE, 1, LANE, blk3, 32, "single")(
        hh3, s3, d3, src_p, dst_p, wb)

    b3p = jnp.concatenate(
        [b3, jnp.zeros((LANE - ncls,), jnp.float32)]).reshape(1, LANE)
    out = _tc_logsoftmax(nd3, b3p, ncls)
    return out[:, :ncls]


# speedup vs baseline: 2.6162x; 2.6162x over previous
import functools

import jax
import jax.numpy as jnp
from jax import lax
from jax.experimental import pallas as pl
from jax.experimental.pallas import tpu as pltpu
from jax.experimental.pallas import tpu_sc as plsc

SC_CORES = 2
SC_SUBCORES = 16
NW = SC_CORES * SC_SUBCORES
LANE = 16


def _lane_bcast(v, h):
    idx = jnp.full((LANE, 1), h, jnp.int32)
    dn = lax.GatherDimensionNumbers(
        offset_dims=(), collapsed_slice_dims=(0,), start_index_map=(0,))
    return lax.gather(v, idx, dn, (1,),
                      mode=lax.GatherScatterMode.PROMISE_IN_BOUNDS)


def _make_edge_pass(n, ep, fr, aw, heads, ch, blk, sub, mode):
    nv_acc = aw // LANE
    mesh = plsc.VectorSubcoreMesh(
        core_axis_name="c", subcore_axis_name="s",
        num_cores=SC_CORES, num_subcores=SC_SUBCORES)

    ep_max = ep - blk

    def body(feat, ssrc, sdst, srcp, dstp, wb, out,
             wbv, sidx0, didx0, dpad0, rows0, srows0, drows0,
             wbuf, acc, gsem0):
        sidx = (sidx0,)
        didx = (didx0,)
        dpad = (dpad0,)
        rows = (rows0,)
        srows = (srows0,)
        drows = (drows0,)
        gsem = (gsem0,)
        cid = lax.axis_index("c")
        sid = lax.axis_index("s")
        wid = sid * SC_CORES + cid
        pltpu.sync_copy(wb.at[wid], wbv)
        wbvec = wbv[...]
        a0 = wbvec[0]
        e_lo = wbvec[1]
        e_hi = wbvec[2]
        d0 = wbvec[3]
        zv = jnp.zeros((LANE,), jnp.float32)
        for v in range(nv_acc):
            acc[pl.ds(LANE * v, LANE)] = zv
        dpad[0][pl.ds(blk, LANE)] = jnp.zeros((LANE,), jnp.int32)

        def e_of(c):
            return pl.multiple_of(jnp.minimum(a0 + c * blk, ep_max), 8)

        def fetch_idx(b, c):
            e0 = e_of(c)
            pltpu.sync_copy(srcp.at[pl.ds(e0, blk)], sidx[b])
            pltpu.sync_copy(dstp.at[pl.ds(e0, blk)], didx[b])
            pltpu.sync_copy(dstp.at[pl.ds(e0, blk)],
                            dpad[b].at[pl.ds(0, blk)])

        def issue(b):
            pltpu.async_copy(feat.at[sidx[b]], rows[b], gsem[b])
            pltpu.async_copy(ssrc.at[sidx[b]], srows[b], gsem[b])
            pltpu.async_copy(sdst.at[didx[b]], drows[b], gsem[b])

        def drain(b):
            pltpu.make_async_copy(feat.at[sidx[b]], rows[b], gsem[b]).wait()
            pltpu.make_async_copy(ssrc.at[sidx[b]], srows[b],
                                  gsem[b]).wait()
            pltpu.make_async_copy(sdst.at[didx[b]], drows[b],
                                  gsem[b]).wait()

        def compute(b, c, cur_d):
            e0r = a0 + c * blk

            def wstep(k, carry):
                e = (srows[b][k, pl.ds(0, LANE)]
                     + drows[b][k, pl.ds(0, LANE)])
                e = jnp.where(e >= 0.0, e, 0.2 * e)
                w = jnp.exp(e)
                ge = e0r + k
                ok = jnp.logical_and(ge >= e_lo, ge < e_hi)
                wbuf[k] = jnp.where(ok, w, 0.0)
                return carry

            lax.fori_loop(0, blk, wstep, 0)

            def estep(k, cur_d):
                dk = dpad[b][pl.ds(k, LANE)][0]
                ge = e0r + k
                ok = jnp.logical_and(ge >= e_lo, ge < e_hi)
                dk = jnp.where(ok, dk, cur_d)

                @pl.when(dk != cur_d)
                def _():
                    pltpu.sync_copy(acc, out.at[cur_d])
                    for v in range(nv_acc):
                        acc[pl.ds(LANE * v, LANE)] = zv

                wv = wbuf[k]
                if mode == "outer":
                    bhs = [_lane_bcast(wv, h) for h in range(heads)]
                    xr = [rows[b][k, pl.ds(LANE * j, LANE)]
                          for j in range(fr // LANE)]
                    for h in range(heads):
                        prods = [bhs[h] * xr[j] for j in range(fr // LANE)]
                        for j in range(fr // LANE):
                            plsc.addupdate(
                                acc.at[pl.ds(h * fr + LANE * j, LANE)],
                                prods[j])
                elif mode == "flat":
                    bhs = [_lane_bcast(wv, h) for h in range(heads)]
                    for h in range(heads):
                        offs = [h * ch + LANE * j
                                for j in range(ch // LANE)]
                        prods = [bhs[h] * rows[b][k, pl.ds(o, LANE)]
                                 for o in offs]
                        for o, p in zip(offs, prods):
                            plsc.addupdate(acc.at[pl.ds(o, LANE)], p)
                else:
                    plsc.addupdate(acc.at[pl.ds(0, LANE)],
                                   wv * rows[b][k, pl.ds(0, LANE)])
                plsc.addupdate(acc.at[pl.ds(aw - LANE, LANE)], wv)
                return dk

            return lax.fori_loop(0, blk, estep, cur_d)

        fetch_idx(0, 0)
        nch = (e_hi - a0 + (blk - 1)) // blk

        def chunkbody(c, cur_d):
            issue(0)
            drain(0)
            cur_d = compute(0, c, cur_d)
            fetch_idx(0, c + 1)
            return cur_d

        cur_d = lax.fori_loop(0, nch, chunkbody, d0)
        pltpu.sync_copy(acc, out.at[cur_d])

    return pl.kernel(
        body,
        out_type=jax.ShapeDtypeStruct((n, aw), jnp.float32),
        mesh=mesh,
        scratch_types=[
            pltpu.VMEM((LANE,), jnp.int32),
            pltpu.VMEM((blk,), jnp.int32),
            pltpu.VMEM((blk,), jnp.int32),
            pltpu.VMEM((blk + LANE,), jnp.int32),
            pltpu.VMEM((blk, fr), jnp.float32),
            pltpu.VMEM((blk, 128), jnp.float32),
            pltpu.VMEM((blk, 128), jnp.float32),
            pltpu.VMEM((blk, LANE), jnp.float32),
            pltpu.VMEM((aw,), jnp.float32),
            pltpu.SemaphoreType.DMA,
        ],
    )


def _prep_edges(edge_index, n, blk_max):
    e = edge_index.shape[1]
    loop = jnp.arange(n, dtype=jnp.int32)
    src = jnp.concatenate([edge_index[0].astype(jnp.int32), loop])
    dst = jnp.concatenate([edge_index[1].astype(jnp.int32), loop])
    order = jnp.argsort(dst)
    src_s = jnp.take(src, order)
    dst_s = jnp.take(dst, order)
    et = e + n
    ep = ((et + blk_max + 7) // 8) * 8
    pad = ep - et
    src_p = jnp.concatenate([src_s, jnp.zeros((pad,), jnp.int32)])
    dst_p = jnp.concatenate([dst_s, jnp.zeros((pad,), jnp.int32)])
    off = jnp.searchsorted(dst_s, jnp.arange(n + 1)).astype(jnp.int32)
    rpw = -(-n // NW)
    wstart = jnp.minimum(jnp.arange(NW, dtype=jnp.int32) * rpw, n)
    wend = jnp.minimum(wstart + rpw, n)
    e_lo = jnp.take(off, wstart)
    e_hi = jnp.take(off, wend)
    a0 = (e_lo // 8) * 8
    wb = jnp.zeros((NW, LANE), jnp.int32)
    wb = (wb.at[:, 0].set(a0).at[:, 1].set(e_lo)
            .at[:, 2].set(e_hi).at[:, 3].set(wstart))
    return src_p, dst_p, wb


def _wcat(w, a_s, a_d):
    din = w.shape[0]
    h, c = a_s.shape
    wr = w.reshape(din, h, c)
    ws = jnp.einsum("dhc,hc->dh", wr, a_s)
    wd = jnp.einsum("dhc,hc->dh", wr, a_d)
    if h == 1:
        return jnp.concatenate(
            [jnp.tile(ws, (1, LANE)), jnp.tile(wd, (1, LANE))], axis=1)
    z = jnp.zeros((din, LANE - h), jnp.float32)
    return jnp.concatenate([ws, z, wd, z], axis=1)


def _tc_scores(x, wcat):
    n, d = x.shape
    br = 512
    grid = pl.cdiv(n, br)

    def body(x_ref, w_ref, s_ref, d_ref):
        s = jnp.dot(x_ref[...], w_ref[...],
                    preferred_element_type=jnp.float32)
        z = jnp.zeros((s.shape[0], 128 - LANE), jnp.float32)
        s_ref[...] = jnp.concatenate([s[:, :LANE], z], axis=1)
        d_ref[...] = jnp.concatenate([s[:, LANE:2 * LANE], z], axis=1)

    return pl.pallas_call(
        body,
        grid=(grid,),
        in_specs=[pl.BlockSpec((br, d), lambda i: (i, 0)),
                  pl.BlockSpec((d, 2 * LANE), lambda i: (0, 0))],
        out_specs=[pl.BlockSpec((br, 128), lambda i: (i, 0)),
                   pl.BlockSpec((br, 128), lambda i: (i, 0))],
        out_shape=[jax.ShapeDtypeStruct((n, 128), jnp.float32)] * 2,
    )(x, wcat)


def _tc_layer1(nd1, w1, b1, g1, be1, w2, wcat2, heads, d_in, c1):
    n = nd1.shape[0]
    f = heads * c1
    hd = heads * d_in
    br = 256
    grid = pl.cdiv(n, br)

    def body(num_ref, den_ref, w1_ref, b1_ref, g1_ref, be1_ref,
             w2_ref, wc_ref, hh_ref, s_ref, d_ref):
        den = den_ref[...][:, :LANE]
        num = num_ref[...]
        parts = []
        for h in range(heads):
            ah = num[:, h * d_in:(h + 1) * d_in] / den[:, h:h + 1]
            parts.append(jnp.dot(ah, w1_ref[:, h * c1:(h + 1) * c1],
                                 preferred_element_type=jnp.float32))
        out1 = jnp.concatenate(parts, axis=1) + b1_ref[...]
        act = jnp.maximum(g1_ref[...] * out1 + be1_ref[...], 0.0)
        hh_ref[...] = jnp.dot(act, w2_ref[...],
                              preferred_element_type=jnp.float32)
        s = jnp.dot(act, wc_ref[...], preferred_element_type=jnp.float32)
        z = jnp.zeros((s.shape[0], 128 - LANE), jnp.float32)
        s_ref[...] = jnp.concatenate([s[:, :LANE], z], axis=1)
        d_ref[...] = jnp.concatenate([s[:, LANE:2 * LANE], z], axis=1)

    return pl.pallas_call(
        body,
        grid=(grid,),
        in_specs=[
            pl.BlockSpec((br, hd), lambda i: (i, 0)),
            pl.BlockSpec((br, 128), lambda i: (i, hd // 128)),
            pl.BlockSpec((d_in, f), lambda i: (0, 0)),
            pl.BlockSpec((1, f), lambda i: (0, 0)),
            pl.BlockSpec((1, f), lambda i: (0, 0)),
            pl.BlockSpec((1, f), lambda i: (0, 0)),
            pl.BlockSpec((f, f), lambda i: (0, 0)),
            pl.BlockSpec((f, 2 * LANE), lambda i: (0, 0)),
        ],
        out_specs=[pl.BlockSpec((br, f), lambda i: (i, 0)),
                   pl.BlockSpec((br, 128), lambda i: (i, 0)),
                   pl.BlockSpec((br, 128), lambda i: (i, 0))],
        out_shape=[jax.ShapeDtypeStruct((n, f), jnp.float32),
                   jax.ShapeDtypeStruct((n, 128), jnp.float32),
                   jax.ShapeDtypeStruct((n, 128), jnp.float32)],
    )(nd1, nd1, w1, b1.reshape(1, f), g1.reshape(1, f),
      be1.reshape(1, f), w2, wcat2)


def _tc_layer2(nd2, b2, g2, be2, w3p, wcat3, heads, c1):
    n = nd2.shape[0]
    f = heads * c1
    br = 256
    grid = pl.cdiv(n, br)

    def body(num_ref, den_ref, b2_ref, g2_ref, be2_ref,
             w3_ref, wc_ref, hh_ref, s_ref, d_ref):
        den = den_ref[...][:, :LANE]
        num = num_ref[...]
        parts = []
        for h in range(heads):
            sl = slice(h * c1, (h + 1) * c1)
            oh = num[:, sl] / den[:, h:h + 1] + b2_ref[:, sl]
            parts.append(jnp.maximum(g2_ref[:, sl] * oh + be2_ref[:, sl],
                                     0.0))
        act = jnp.concatenate(parts, axis=1)
        hh_ref[...] = jnp.dot(act, w3_ref[...],
                              preferred_element_type=jnp.float32)
        s = jnp.dot(act, wc_ref[...], preferred_element_type=jnp.float32)
        z = jnp.zeros((s.shape[0], 128 - LANE), jnp.float32)
        s_ref[...] = jnp.concatenate([s[:, :LANE], z], axis=1)
        d_ref[...] = jnp.concatenate([s[:, LANE:2 * LANE], z], axis=1)

    return pl.pallas_call(
        body,
        grid=(grid,),
        in_specs=[
            pl.BlockSpec((br, f), lambda i: (i, 0)),
            pl.BlockSpec((br, 128), lambda i: (i, f // 128)),
            pl.BlockSpec((1, f), lambda i: (0, 0)),
            pl.BlockSpec((1, f), lambda i: (0, 0)),
            pl.BlockSpec((1, f), lambda i: (0, 0)),
            pl.BlockSpec((f, 128), lambda i: (0, 0)),
            pl.BlockSpec((f, 2 * LANE), lambda i: (0, 0)),
        ],
        out_specs=[pl.BlockSpec((br, 128), lambda i: (i, 0)),
                   pl.BlockSpec((br, 128), lambda i: (i, 0)),
                   pl.BlockSpec((br, 128), lambda i: (i, 0))],
        out_shape=[jax.ShapeDtypeStruct((n, 128), jnp.float32),
                   jax.ShapeDtypeStruct((n, 128), jnp.float32),
                   jax.ShapeDtypeStruct((n, 128), jnp.float32)],
    )(nd2, nd2, b2.reshape(1, f), g2.reshape(1, f), be2.reshape(1, f),
      w3p, wcat3)


def _tc_logsoftmax(nd3, b3p, ncls):
    n = nd3.shape[0]
    br = 512
    grid = pl.cdiv(n, br)

    def body(nd_ref, b3_ref, out_ref):
        nd = nd_ref[...]
        h = nd[:, :LANE] / nd[:, LANE:2 * LANE] + b3_ref[...]
        col = lax.broadcasted_iota(jnp.int32, h.shape, 1)
        valid = col < ncls
        hm = jnp.where(valid, h, jnp.float32(-1e30))
        m = jnp.max(hm, axis=1, keepdims=True)
        ex = jnp.where(valid, jnp.exp(h - m), 0.0)
        s = jnp.sum(ex, axis=1, keepdims=True)
        out_ref[...] = h - m - jnp.log(s)

    return pl.pallas_call(
        body,
        grid=(grid,),
        in_specs=[pl.BlockSpec((br, 2 * LANE), lambda i: (i, 0)),
                  pl.BlockSpec((1, LANE), lambda i: (0, 0))],
        out_specs=pl.BlockSpec((br, LANE), lambda i: (i, 0)),
        out_shape=jax.ShapeDtypeStruct((n, LANE), jnp.float32),
    )(nd3, b3p)


@jax.jit
def kernel(x, edge_index, W1, a1s, a1d, b1, g1, be1,
           W2, a2s, a2d, b2, g2, be2, W3, a3s, a3d, b3):
    n, d = x.shape
    heads, c1 = a1s.shape
    f = heads * c1
    ncls = W3.shape[1]

    blk1, blk2, blk3 = 128, 48, 128
    src_p, dst_p, wb = _prep_edges(edge_index, n, max(blk1, blk2, blk3))
    ep = src_p.shape[0]

    s1, d1 = _tc_scores(x, _wcat(W1, a1s, a1d))
    aw1 = heads * d + LANE
    nd1 = _make_edge_pass(n, ep, d, aw1, heads, d, blk1, 16, "outer")(
        x, s1, d1, src_p, dst_p, wb)

    hh2, s2, d2 = _tc_layer1(nd1, W1, b1, g1, be1, W2,
                             _wcat(W2, a2s, a2d), heads, d, c1)

    aw2 = f + LANE
    nd2 = _make_edge_pass(n, ep, f, aw2, heads, c1, blk2, 8, "flat")(
        hh2, s2, d2, src_p, dst_p, wb)

    w3p = jnp.concatenate(
        [W3, jnp.zeros((f, 128 - ncls), jnp.float32)], axis=1)
    hh3, s3, d3 = _tc_layer2(nd2, b2, g2, be2, w3p,
                             _wcat(W3, a3s, a3d), heads, c1)

    nd3 = _make_edge_pass(n, ep, 128, 2 * LANE, 1, LANE, blk3, 32, "single")(
        hh3, s3, d3, src_p, dst_p, wb)

    b3p = jnp.concatenate(
        [b3, jnp.zeros((LANE - ncls,), jnp.float32)]).reshape(1, LANE)
    out = _tc_logsoftmax(nd3, b3p, ncls)
    return out[:, :ncls]
